# baseline (speedup 1.0000x reference)
import functools

import jax
import jax.numpy as jnp
from jax import lax
from jax.experimental import pallas as pl
from jax.experimental.pallas import tpu as pltpu
from jax.experimental.pallas import tpu_sc as plsc

_B = 16384
_L = 200
_EMB = 32
_HID = 128

_NC = 2
_NS = 16
_NW = _NC * _NS
_ROWS_PW = _B // _NW
_CB = 8
_NCHUNK = _ROWS_PW // _CB
_LA = 128
_LB = _L - _LA


_RU = 8
_NACC = 4


def _make_sc_pool():
  mesh = plsc.VectorSubcoreMesh(core_axis_name="c", subcore_axis_name="s")

  @functools.partial(
      pl.kernel,
      mesh=mesh,
      out_type=jax.ShapeDtypeStruct((_B, _EMB), jnp.float32),
      compiler_params=pltpu.CompilerParams(use_tc_tiling_on_sc=False),
      scratch_types=[
          pltpu.VMEM((2, _CB * _L), jnp.int32),
          pltpu.VMEM((2, _CB * _L, _EMB), jnp.float32),
          pltpu.VMEM((2, _CB, _EMB), jnp.float32),
          pltpu.SemaphoreType.DMA,
          pltpu.SemaphoreType.DMA,
          pltpu.SemaphoreType.DMA,
          pltpu.SemaphoreType.DMA,
          pltpu.SemaphoreType.DMA,
          pltpu.SemaphoreType.DMA,
      ],
  )
  def sc_pool(xf_hbm, table_hbm, sums_hbm, idx_v, rows_v, out_v, sem0, sem1,
              isem0, isem1, osem0, osem1):
    wid = lax.axis_index("s") * _NC + lax.axis_index("c")
    base = wid * _ROWS_PW
    sems = (sem0, sem1)
    isems = (isem0, isem1)
    osems = (osem0, osem1)

    def gather_descs(b):
      iv = idx_v.at[b]
      descs = []
      for r in range(_CB):
        descs.append(pltpu.make_async_copy(
            table_hbm.at[iv.at[pl.ds(r * _L, _LA)]],
            rows_v.at[b].at[pl.ds(r * _L, _LA)], sems[b]))
        descs.append(pltpu.make_async_copy(
            table_hbm.at[iv.at[pl.ds(r * _L + _LA, _LB)]],
            rows_v.at[b].at[pl.ds(r * _L + _LA, _LB)], sems[b]))
      return descs

    def idx_desc(g, b):
      rbase = base + g * _CB
      return pltpu.make_async_copy(
          xf_hbm.at[pl.ds(rbase * _L, _CB * _L)], idx_v.at[b], isems[b])

    def start_gathers(b):
      for d in gather_descs(b):
        d.start()

    def drain(b):
      for d in gather_descs(b):
        d.wait()

    def out_desc(g, b):
      rbase = base + g * _CB
      return pltpu.make_async_copy(
          out_v.at[b], sums_hbm.at[pl.ds(rbase, _CB)], osems[b])

    def reduce_store(g, b):
      out_desc(g, b).wait()
      rv = rows_v.at[b]
      ov = out_v.at[b]
      for r in range(_CB):
        rowbase = r * _L

        def red_body(i, acc, rowbase=rowbase, rv=rv):
          accs = list(acc)
          rb = rowbase + i * _RU
          for j in range(_RU):
            k = j % _NACC
            accs[2 * k] = accs[2 * k] + rv[rb + j, pl.ds(0, 16)]
            accs[2 * k + 1] = accs[2 * k + 1] + rv[rb + j, pl.ds(16, 16)]
          return tuple(accs)

        z = jnp.zeros((16,), jnp.float32)
        acc = lax.fori_loop(0, _L // _RU, red_body, (z,) * (2 * _NACC))
        a0 = (acc[0] + acc[2]) + (acc[4] + acc[6])
        a1 = (acc[1] + acc[3]) + (acc[5] + acc[7])
        ov[r, pl.ds(0, 16)] = a0
        ov[r, pl.ds(16, 16)] = a1
      out_desc(g, b).start()

    idx_desc(0, 0).start()
    out_desc(0, 0).start()
    out_desc(1, 1).start()
    idx_desc(0, 0).wait()
    start_gathers(0)
    idx_desc(1, 1).start()

    def body2(h, carry):
      g0 = 2 * h
      idx_desc(0, 1).wait()
      start_gathers(1)
      drain(0)
      idx_desc(lax.rem(g0 + 2, _NCHUNK), 0).start()
      reduce_store(g0, 0)
      drain(1)
      idx_desc(0, 0).wait()
      start_gathers(0)
      idx_desc(lax.rem(g0 + 3, _NCHUNK), 1).start()
      reduce_store(g0 + 1, 1)
      return carry

    lax.fori_loop(0, _NCHUNK // 2, body2, 0)
    drain(0)
    idx_desc(0, 1).wait()
    out_desc(0, 0).wait()
    out_desc(0, 1).wait()

  return sc_pool


_sc_pool = _make_sc_pool()


_BT = 2048


def _tc_body(x_ref, sums_ref, w1_ref, b1_ref, w2_ref, b2_ref, out_ref):
  xm = (x_ref[...] != 0).astype(jnp.float32)
  cnt = jnp.sum(xm, axis=1, keepdims=True)
  cnt = jnp.maximum(cnt, 1e-9)
  mean = sums_ref[...] / cnt
  h = jnp.dot(mean, w1_ref[...], preferred_element_type=jnp.float32)
  h = jnp.maximum(h + b1_ref[...], 0.0)
  out_ref[...] = (
      jnp.dot(h, w2_ref[...], preferred_element_type=jnp.float32)
      + b2_ref[...])


def _tc_mlp(x, sums, W1, b1, W2, b2):
  grid = (_B // _BT,)
  return pl.pallas_call(
      _tc_body,
      grid=grid,
      in_specs=[
          pl.BlockSpec((_BT, _L), lambda i: (i, 0)),
          pl.BlockSpec((_BT, _EMB), lambda i: (i, 0)),
          pl.BlockSpec((_EMB, _HID), lambda i: (0, 0)),
          pl.BlockSpec((1, _HID), lambda i: (0, 0)),
          pl.BlockSpec((_HID, 2), lambda i: (0, 0)),
          pl.BlockSpec((1, 2), lambda i: (0, 0)),
      ],
      out_specs=pl.BlockSpec((_BT, 2), lambda i: (i, 0)),
      out_shape=jax.ShapeDtypeStruct((_B, 2), jnp.float32),
  )(x, sums, W1, b1.reshape(1, _HID), W2, b2.reshape(1, 2))


def kernel(x, table, W1, b1, W2, b2):
  x = x.astype(jnp.int32)
  xf = x.reshape(_B * _L)
  sums = _sc_pool(xf, table)
  return _tc_mlp(x, sums, W1, b1, W2, b2)


# speedup vs baseline: 1.0026x; 1.0026x over previous
import functools

import jax
import jax.numpy as jnp
from jax import lax
from jax.experimental import pallas as pl
from jax.experimental.pallas import tpu as pltpu
from jax.experimental.pallas import tpu_sc as plsc

_B = 16384
_L = 200
_EMB = 32
_HID = 128

_NC = 2
_NS = 16
_NW = _NC * _NS
_ROWS_PW = _B // _NW
_CB = 8
_NCHUNK = _ROWS_PW // _CB
_LA = 128
_LB = _L - _LA


_RU = 8
_NACC = 4


def _make_sc_pool():
  mesh = plsc.VectorSubcoreMesh(core_axis_name="c", subcore_axis_name="s")

  @functools.partial(
      pl.kernel,
      mesh=mesh,
      out_type=jax.ShapeDtypeStruct((_B, _EMB), jnp.float32),
      compiler_params=pltpu.CompilerParams(use_tc_tiling_on_sc=False),
      scratch_types=[
          pltpu.VMEM((2, _CB * _L), jnp.int32),
          pltpu.VMEM((2, _CB * _L, _EMB), jnp.float32),
          pltpu.VMEM((2, _CB, _EMB), jnp.float32),
          pltpu.SemaphoreType.DMA,
          pltpu.SemaphoreType.DMA,
          pltpu.SemaphoreType.DMA,
          pltpu.SemaphoreType.DMA,
          pltpu.SemaphoreType.DMA,
          pltpu.SemaphoreType.DMA,
      ],
  )
  def sc_pool(xf_hbm, table_hbm, sums_hbm, idx_v, rows_v, out_v, sem0, sem1,
              isem0, isem1, osem0, osem1):
    wid = lax.axis_index("s") * _NC + lax.axis_index("c")
    base = wid * _ROWS_PW
    sems = (sem0, sem1)
    isems = (isem0, isem1)
    osems = (osem0, osem1)

    def gather_descs(b):
      iv = idx_v.at[b]
      descs = []
      total = _CB * _L
      for s in range(0, total, _LA):
        n = min(_LA, total - s)
        descs.append(pltpu.make_async_copy(
            table_hbm.at[iv.at[pl.ds(s, n)]],
            rows_v.at[b].at[pl.ds(s, n)], sems[b]))
      return descs

    def idx_desc(g, b):
      rbase = base + g * _CB
      return pltpu.make_async_copy(
          xf_hbm.at[pl.ds(rbase * _L, _CB * _L)], idx_v.at[b], isems[b])

    def start_gathers(b):
      for d in gather_descs(b):
        d.start()

    def drain(b):
      for d in gather_descs(b):
        d.wait()

    def out_desc(g, b):
      rbase = base + g * _CB
      return pltpu.make_async_copy(
          out_v.at[b], sums_hbm.at[pl.ds(rbase, _CB)], osems[b])

    def reduce_store(g, b):
      out_desc(g, b).wait()
      rv = rows_v.at[b]
      ov = out_v.at[b]
      for r in range(_CB):
        rowbase = r * _L

        def red_body(i, acc, rowbase=rowbase, rv=rv):
          accs = list(acc)
          rb = rowbase + i * _RU
          for j in range(_RU):
            k = j % _NACC
            accs[2 * k] = accs[2 * k] + rv[rb + j, pl.ds(0, 16)]
            accs[2 * k + 1] = accs[2 * k + 1] + rv[rb + j, pl.ds(16, 16)]
          return tuple(accs)

        z = jnp.zeros((16,), jnp.float32)
        acc = lax.fori_loop(0, _L // _RU, red_body, (z,) * (2 * _NACC))
        a0 = (acc[0] + acc[2]) + (acc[4] + acc[6])
        a1 = (acc[1] + acc[3]) + (acc[5] + acc[7])
        ov[r, pl.ds(0, 16)] = a0
        ov[r, pl.ds(16, 16)] = a1
      out_desc(g, b).start()

    idx_desc(0, 0).start()
    out_desc(0, 0).start()
    out_desc(1, 1).start()
    idx_desc(0, 0).wait()
    start_gathers(0)
    idx_desc(1, 1).start()

    def body2(h, carry):
      g0 = 2 * h
      idx_desc(0, 1).wait()
      start_gathers(1)
      drain(0)
      idx_desc(lax.rem(g0 + 2, _NCHUNK), 0).start()
      reduce_store(g0, 0)
      drain(1)
      idx_desc(0, 0).wait()
      start_gathers(0)
      idx_desc(lax.rem(g0 + 3, _NCHUNK), 1).start()
      reduce_store(g0 + 1, 1)
      return carry

    lax.fori_loop(0, _NCHUNK // 2, body2, 0)
    drain(0)
    idx_desc(0, 1).wait()
    out_desc(0, 0).wait()
    out_desc(0, 1).wait()

  return sc_pool


_sc_pool = _make_sc_pool()


_BT = 2048


def _tc_body(x_ref, sums_ref, w1_ref, b1_ref, w2_ref, b2_ref, out_ref):
  xm = (x_ref[...] != 0).astype(jnp.float32)
  cnt = jnp.sum(xm, axis=1, keepdims=True)
  cnt = jnp.maximum(cnt, 1e-9)
  mean = sums_ref[...] / cnt
  h = jnp.dot(mean, w1_ref[...], preferred_element_type=jnp.float32)
  h = jnp.maximum(h + b1_ref[...], 0.0)
  out_ref[...] = (
      jnp.dot(h, w2_ref[...], preferred_element_type=jnp.float32)
      + b2_ref[...])


def _tc_mlp(x, sums, W1, b1, W2, b2):
  grid = (_B // _BT,)
  return pl.pallas_call(
      _tc_body,
      grid=grid,
      in_specs=[
          pl.BlockSpec((_BT, _L), lambda i: (i, 0)),
          pl.BlockSpec((_BT, _EMB), lambda i: (i, 0)),
          pl.BlockSpec((_EMB, _HID), lambda i: (0, 0)),
          pl.BlockSpec((1, _HID), lambda i: (0, 0)),
          pl.BlockSpec((_HID, 2), lambda i: (0, 0)),
          pl.BlockSpec((1, 2), lambda i: (0, 0)),
      ],
      out_specs=pl.BlockSpec((_BT, 2), lambda i: (i, 0)),
      out_shape=jax.ShapeDtypeStruct((_B, 2), jnp.float32),
  )(x, sums, W1, b1.reshape(1, _HID), W2, b2.reshape(1, 2))


def kernel(x, table, W1, b1, W2, b2):
  x = x.astype(jnp.int32)
  xf = x.reshape(_B * _L)
  sums = _sc_pool(xf, table)
  return _tc_mlp(x, sums, W1, b1, W2, b2)
